# baseline (speedup 1.0000x reference)
import functools

import jax
import jax.numpy as jnp
from jax import lax
from jax.experimental import pallas as pl
from jax.experimental.pallas import tpu as pltpu
from jax.experimental.pallas import tpu_sc as plsc

N = 100000
D = 128
H = 64
B = 512

R = 16384
NB = -(-(N // 8) // (R // 8))
NW = 32
CHUNK = N // NW
CHUNK16 = ((CHUNK + 15) // 16) * 16


def _mlp_body(x_ref, ids_ref, w1_ref, b1_ref, w2_ref, b2_ref, oe_ref, oi_ref):
    x = x_ref[...]
    h = jnp.dot(x, w1_ref[...], preferred_element_type=jnp.float32)
    h = h + b1_ref[...]
    h = h * jax.nn.sigmoid(h)
    e = jnp.dot(h, w2_ref[...], preferred_element_type=jnp.float32)
    oe_ref[...] = jnp.transpose((e + b2_ref[...]).reshape(R // 8, 8))
    oi_ref[...] = jnp.transpose(ids_ref[...])


_mlp = pl.pallas_call(
    _mlp_body,
    grid=(NB,),
    in_specs=[
        pl.BlockSpec((R, D), lambda i: (i, 0)),
        pl.BlockSpec((R // 8, 8), lambda i: (i, 0)),
        pl.BlockSpec((D, H), lambda i: (0, 0)),
        pl.BlockSpec((1, H), lambda i: (0, 0)),
        pl.BlockSpec((H, 1), lambda i: (0, 0)),
        pl.BlockSpec((1, 1), lambda i: (0, 0)),
    ],
    out_specs=[
        pl.BlockSpec((8, R // 8), lambda i: (0, i)),
        pl.BlockSpec((8, R // 8), lambda i: (0, i)),
    ],
    out_shape=[
        jax.ShapeDtypeStruct((8, N // 8), jnp.float32),
        jax.ShapeDtypeStruct((8, N // 8), jnp.int32),
    ],
)

_mesh = plsc.VectorSubcoreMesh(core_axis_name="c", subcore_axis_name="s")


@functools.partial(
    pl.kernel,
    out_type=(
        jax.ShapeDtypeStruct((2, B), jnp.float32),
        jax.ShapeDtypeStruct((2, B), jnp.float32),
    ),
    mesh=_mesh,
    scratch_types=[
        pltpu.VMEM((CHUNK,), jnp.float32),
        pltpu.VMEM((CHUNK,), jnp.int32),
        pltpu.VMEM((CHUNK16,), jnp.float32),
        pltpu.VMEM((B,), jnp.float32),
        pltpu.VMEM_SHARED((B,), jnp.float32),
        pltpu.VMEM_SHARED((B,), jnp.float32),
    ],
)
def _sc_segsum(e_hbm, idx_hbm, out_e, out_c,
               e_v, idx_v, ones_v, zeros_v, acc_e, acc_c):
    cid = lax.axis_index("c")
    sid = lax.axis_index("s")
    wid = cid * 16 + sid

    pltpu.sync_copy(e_hbm.at[wid], e_v)
    pltpu.sync_copy(idx_hbm.at[wid], idx_v)

    def fill_ones(i, _):
        for u in range(4):
            ones_v[pl.ds(i * 64 + u * 16, 16)] = jnp.ones((16,), jnp.float32)
        return 0
    lax.fori_loop(0, CHUNK16 // 64, fill_ones, 0)

    @pl.when(sid == 0)
    def _():
        def fill_zeros(i, _):
            for u in range(4):
                zeros_v[pl.ds(i * 64 + u * 16, 16)] = jnp.zeros((16,), jnp.float32)
            return 0
        lax.fori_loop(0, B // 64, fill_zeros, 0)
        pltpu.sync_copy(zeros_v, acc_e)
        pltpu.sync_copy(zeros_v, acc_c)

    plsc.subcore_barrier()
    pltpu.sync_copy(e_v, acc_e.at[idx_v], add=True)
    pltpu.sync_copy(ones_v.at[pl.ds(0, CHUNK)], acc_c.at[idx_v], add=True)
    plsc.subcore_barrier()

    @pl.when(sid == 0)
    def _():
        pltpu.sync_copy(acc_e, out_e.at[cid])
        pltpu.sync_copy(acc_c, out_c.at[cid])


def kernel(node_feats, pos, batch, W1, b1, W2, b2):
    batch2d = batch.astype(jnp.int32).reshape(N // 8, 8)
    e8, idx8 = _mlp(node_feats, batch2d, W1, b1.reshape(1, H), W2,
                    b2.reshape(1, 1))
    e32 = e8.reshape(NW, CHUNK)
    idx32 = idx8.reshape(NW, CHUNK)
    out_e, out_c = _sc_segsum(e32, idx32)

    energy = out_e[0] + out_e[1]
    num_atoms = out_c[0] + out_c[1]
    forces = jnp.zeros_like(pos)
    return (energy, forces, num_atoms)


# speedup vs baseline: 1.0267x; 1.0267x over previous
import functools

import jax
import jax.numpy as jnp
from jax import lax
from jax.experimental import pallas as pl
from jax.experimental.pallas import tpu as pltpu
from jax.experimental.pallas import tpu_sc as plsc

N = 100000
D = 128
H = 64
B = 512

R = 13312
NB = -(-(N // 8) // (R // 8))
NW = 32
CHUNK = N // NW
CHUNK16 = ((CHUNK + 15) // 16) * 16


def _mlp_body(x_ref, ids_ref, w1_ref, b1_ref, w2_ref, b2_ref, oe_ref, oi_ref):
    x = x_ref[...]
    h = jnp.dot(x, w1_ref[...], preferred_element_type=jnp.float32)
    h = h + b1_ref[...]
    h = h * jax.nn.sigmoid(h)
    e = jnp.dot(h, w2_ref[...], preferred_element_type=jnp.float32)
    oe_ref[...] = jnp.transpose((e + b2_ref[...]).reshape(R // 8, 8))
    oi_ref[...] = jnp.transpose(ids_ref[...])


_mlp = pl.pallas_call(
    _mlp_body,
    grid=(NB,),
    in_specs=[
        pl.BlockSpec((R, D), lambda i: (i, 0)),
        pl.BlockSpec((R // 8, 8), lambda i: (i, 0)),
        pl.BlockSpec((D, H), lambda i: (0, 0)),
        pl.BlockSpec((1, H), lambda i: (0, 0)),
        pl.BlockSpec((H, 1), lambda i: (0, 0)),
        pl.BlockSpec((1, 1), lambda i: (0, 0)),
    ],
    out_specs=[
        pl.BlockSpec((8, R // 8), lambda i: (0, i)),
        pl.BlockSpec((8, R // 8), lambda i: (0, i)),
    ],
    out_shape=[
        jax.ShapeDtypeStruct((8, N // 8), jnp.float32),
        jax.ShapeDtypeStruct((8, N // 8), jnp.int32),
    ],
)

_mesh = plsc.VectorSubcoreMesh(core_axis_name="c", subcore_axis_name="s")


@functools.partial(
    pl.kernel,
    out_type=(
        jax.ShapeDtypeStruct((2, B), jnp.float32),
        jax.ShapeDtypeStruct((2, B), jnp.float32),
    ),
    mesh=_mesh,
    scratch_types=[
        pltpu.VMEM((CHUNK,), jnp.float32),
        pltpu.VMEM((CHUNK,), jnp.int32),
        pltpu.VMEM((CHUNK16,), jnp.float32),
        pltpu.VMEM((B,), jnp.float32),
        pltpu.VMEM_SHARED((B,), jnp.float32),
        pltpu.VMEM_SHARED((B,), jnp.float32),
    ],
)
def _sc_segsum(e_hbm, idx_hbm, out_e, out_c,
               e_v, idx_v, ones_v, zeros_v, acc_e, acc_c):
    cid = lax.axis_index("c")
    sid = lax.axis_index("s")
    wid = cid * 16 + sid

    pltpu.sync_copy(e_hbm.at[wid], e_v)
    pltpu.sync_copy(idx_hbm.at[wid], idx_v)

    def fill_ones(i, _):
        for u in range(4):
            ones_v[pl.ds(i * 64 + u * 16, 16)] = jnp.ones((16,), jnp.float32)
        return 0
    lax.fori_loop(0, CHUNK16 // 64, fill_ones, 0)

    @pl.when(sid == 0)
    def _():
        def fill_zeros(i, _):
            for u in range(4):
                zeros_v[pl.ds(i * 64 + u * 16, 16)] = jnp.zeros((16,), jnp.float32)
            return 0
        lax.fori_loop(0, B // 64, fill_zeros, 0)
        pltpu.sync_copy(zeros_v, acc_e)
        pltpu.sync_copy(zeros_v, acc_c)

    plsc.subcore_barrier()
    pltpu.sync_copy(e_v, acc_e.at[idx_v], add=True)
    pltpu.sync_copy(ones_v.at[pl.ds(0, CHUNK)], acc_c.at[idx_v], add=True)
    plsc.subcore_barrier()

    @pl.when(sid == 0)
    def _():
        pltpu.sync_copy(acc_e, out_e.at[cid])
        pltpu.sync_copy(acc_c, out_c.at[cid])


def kernel(node_feats, pos, batch, W1, b1, W2, b2):
    batch2d = batch.astype(jnp.int32).reshape(N // 8, 8)
    e8, idx8 = _mlp(node_feats, batch2d, W1, b1.reshape(1, H), W2,
                    b2.reshape(1, 1))
    e32 = e8.reshape(NW, CHUNK)
    idx32 = idx8.reshape(NW, CHUNK)
    out_e, out_c = _sc_segsum(e32, idx32)

    energy = out_e[0] + out_e[1]
    num_atoms = out_c[0] + out_c[1]
    forces = jnp.zeros_like(pos)
    return (energy, forces, num_atoms)
